# baseline (speedup 1.0000x reference)
import functools

import jax
import jax.numpy as jnp
from jax import lax
from jax.experimental import pallas as pl
from jax.experimental.pallas import tpu as pltpu

_BATCH = 16384
_DIM = 64


def _body(labels_smem, table_hbm, out_vmem, sem):
    def fire(i, c):
        row = labels_smem[i]
        pltpu.make_async_copy(
            table_hbm.at[pl.ds(row, 1), :],
            out_vmem.at[pl.ds(i, 1), :],
            sem,
        ).start()
        return c

    lax.fori_loop(0, _BATCH, fire, 0)
    pltpu.make_async_copy(
        table_hbm.at[pl.ds(0, _BATCH), :],
        out_vmem,
        sem,
    ).wait()


@jax.jit
def kernel(labels, table):
    grid_spec = pltpu.PrefetchScalarGridSpec(
        num_scalar_prefetch=1,
        grid=(1,),
        in_specs=[pl.BlockSpec(memory_space=pltpu.HBM)],
        out_specs=pl.BlockSpec(memory_space=pltpu.VMEM),
        scratch_shapes=[pltpu.SemaphoreType.DMA],
    )
    return pl.pallas_call(
        _body,
        grid_spec=grid_spec,
        out_shape=jax.ShapeDtypeStruct((_BATCH, _DIM), jnp.float32),
    )(labels.astype(jnp.int32), table)
